# baseline (speedup 1.0000x reference)
import jax
import jax.numpy as jnp
from jax.experimental import pallas as pl
from jax.experimental.pallas import tpu as pltpu

_B, _N, _C = 32, 8732, 21
_H = _B // 2
_CPB = 3
_STEPS = _C // _CPB


def _isd_body(lam_ref, x_ref, s_ref, i_ref, out_ref,
              acc_ref, x0_ref, t0_ref, lmax_ref, rmax_ref):
    g = pl.program_id(0)
    lam = lam_ref[0]

    for p in range(_CPB):
        for r in range(2):
            rows = pl.ds((2 * p + r) * _H, _H)
            srows = pl.ds((2 * p + (1 - r)) * _H, _H)
            sl = pl.ds(r * _H, _H)

            x = x_ref[rows, :]
            t = s_ref[srows, :]
            ci = i_ref[rows, :]

            mixed = lam * x + ((1.0 - lam) * t + 1e-7)
            interp = ci + 1e-7
            contrib = (interp - mixed) * (jnp.log2(interp) - jnp.log2(mixed))

            if p == 0:
                @pl.when(g == 0)
                def _(contrib=contrib, x=x, t=t, sl=sl):
                    acc_ref[sl, :] = contrib
                    x0_ref[sl, :] = x
                    t0_ref[sl, :] = t

                @pl.when(g > 0)
                def _(contrib=contrib, x=x, t=t, sl=sl):
                    acc_ref[sl, :] += contrib
                    lmax_ref[sl, :] = jnp.maximum(lmax_ref[sl, :], x)
                    rmax_ref[sl, :] = jnp.maximum(rmax_ref[sl, :], t)
            elif p == 1:
                acc_ref[sl, :] += contrib

                @pl.when(g == 0)
                def _(x=x, t=t, sl=sl):
                    lmax_ref[sl, :] = x
                    rmax_ref[sl, :] = t

                @pl.when(g > 0)
                def _(x=x, t=t, sl=sl):
                    lmax_ref[sl, :] = jnp.maximum(lmax_ref[sl, :], x)
                    rmax_ref[sl, :] = jnp.maximum(rmax_ref[sl, :], t)
            else:
                acc_ref[sl, :] += contrib
                lmax_ref[sl, :] = jnp.maximum(lmax_ref[sl, :], x)
                rmax_ref[sl, :] = jnp.maximum(rmax_ref[sl, :], t)

    @pl.when(g == _STEPS - 1)
    def _():
        mask = jnp.logical_and(lmax_ref[...] > x0_ref[...],
                               rmax_ref[...] > t0_ref[...])
        maskf = mask.astype(jnp.float32)
        s = jnp.sum(acc_ref[...] * maskf) * 0.6931471805599453
        cnt = jnp.sum(maskf)
        loss = 0.5 * s / jnp.maximum(cnt, 1.0)
        out_ref[0] = jnp.where(cnt > 0.0, loss, 0.0)


@jax.jit
def _isd_loss(lam, conf, conf_shuffle, conf_interpolation):
    ct = jnp.transpose(conf, (2, 0, 1)).reshape(_C * _B, _N)
    st = jnp.transpose(conf_shuffle, (2, 0, 1)).reshape(_C * _B, _N)
    it = jnp.transpose(conf_interpolation, (2, 0, 1)).reshape(_C * _B, _N)
    blk = _CPB * _B
    out = pl.pallas_call(
        _isd_body,
        grid=(_STEPS,),
        in_specs=[
            pl.BlockSpec(memory_space=pltpu.SMEM),
            pl.BlockSpec((blk, _N), lambda g: (g, 0)),
            pl.BlockSpec((blk, _N), lambda g: (g, 0)),
            pl.BlockSpec((blk, _N), lambda g: (g, 0)),
        ],
        out_specs=pl.BlockSpec(memory_space=pltpu.SMEM),
        out_shape=jax.ShapeDtypeStruct((1,), jnp.float32),
        scratch_shapes=[
            pltpu.VMEM((_B, _N), jnp.float32),
            pltpu.VMEM((_B, _N), jnp.float32),
            pltpu.VMEM((_B, _N), jnp.float32),
            pltpu.VMEM((_B, _N), jnp.float32),
            pltpu.VMEM((_B, _N), jnp.float32),
        ],
        compiler_params=pltpu.CompilerParams(
            dimension_semantics=("arbitrary",),
        ),
    )(jnp.asarray(lam, jnp.float32).reshape(1), ct, st, it)
    return out[0]


def kernel(lam, conf, conf_flip, loc, loc_flip, conf_shuffle,
           conf_interpolation, loc_shuffle, loc_interpolation):
    return _isd_loss(lam, conf, conf_shuffle, conf_interpolation)


# speedup vs baseline: 1.0870x; 1.0870x over previous
import jax
import jax.numpy as jnp
from jax.experimental import pallas as pl
from jax.experimental.pallas import tpu as pltpu

_B, _N, _C = 32, 8732, 21
_H = _B // 2
_CPB = 3
_STEPS = _C // _CPB


def _isd_body(lam_ref, x_ref, s_ref, i_ref, out_ref,
              acc_ref, x0_ref, t0_ref, lmax_ref, rmax_ref):
    g = pl.program_id(0)
    lam = lam_ref[0]

    ncols = [(0, 2304), (2304, 2304), (4608, 2304), (6912, _N - 6912)]
    for p in range(_CPB):
        for r in range(2):
            rows = pl.ds((2 * p + r) * _H, _H)
            srows = pl.ds((2 * p + (1 - r)) * _H, _H)
            sl = pl.ds(r * _H, _H)
            for c0, cw in ncols:
                cs = pl.ds(c0, cw)

                x = x_ref[rows, cs]
                t = s_ref[srows, cs]
                ci = i_ref[rows, cs]

                mixed = lam * x + ((1.0 - lam) * t + 1e-7)
                interp = ci + 1e-7
                contrib = (interp - mixed) * (jnp.log(interp)
                                              - jnp.log(mixed))

                if p == 0:
                    @pl.when(g == 0)
                    def _(contrib=contrib, x=x, t=t, sl=sl, cs=cs):
                        acc_ref[sl, cs] = contrib
                        x0_ref[sl, cs] = x
                        t0_ref[sl, cs] = t

                    @pl.when(g > 0)
                    def _(contrib=contrib, x=x, t=t, sl=sl, cs=cs):
                        acc_ref[sl, cs] += contrib
                        lmax_ref[sl, cs] = jnp.maximum(lmax_ref[sl, cs], x)
                        rmax_ref[sl, cs] = jnp.maximum(rmax_ref[sl, cs], t)
                elif p == 1:
                    acc_ref[sl, cs] += contrib

                    @pl.when(g == 0)
                    def _(x=x, t=t, sl=sl, cs=cs):
                        lmax_ref[sl, cs] = x
                        rmax_ref[sl, cs] = t

                    @pl.when(g > 0)
                    def _(x=x, t=t, sl=sl, cs=cs):
                        lmax_ref[sl, cs] = jnp.maximum(lmax_ref[sl, cs], x)
                        rmax_ref[sl, cs] = jnp.maximum(rmax_ref[sl, cs], t)
                else:
                    acc_ref[sl, cs] += contrib
                    lmax_ref[sl, cs] = jnp.maximum(lmax_ref[sl, cs], x)
                    rmax_ref[sl, cs] = jnp.maximum(rmax_ref[sl, cs], t)

    @pl.when(g == _STEPS - 1)
    def _():
        mask = jnp.logical_and(lmax_ref[...] > x0_ref[...],
                               rmax_ref[...] > t0_ref[...])
        maskf = mask.astype(jnp.float32)
        s = jnp.sum(acc_ref[...] * maskf)
        cnt = jnp.sum(maskf)
        loss = 0.5 * s / jnp.maximum(cnt, 1.0)
        out_ref[0] = jnp.where(cnt > 0.0, loss, 0.0)


@jax.jit
def _isd_loss(lam, conf, conf_shuffle, conf_interpolation):
    ct = jnp.transpose(conf, (2, 0, 1)).reshape(_C * _B, _N)
    st = jnp.transpose(conf_shuffle, (2, 0, 1)).reshape(_C * _B, _N)
    it = jnp.transpose(conf_interpolation, (2, 0, 1)).reshape(_C * _B, _N)
    blk = _CPB * _B
    out = pl.pallas_call(
        _isd_body,
        grid=(_STEPS,),
        in_specs=[
            pl.BlockSpec(memory_space=pltpu.SMEM),
            pl.BlockSpec((blk, _N), lambda g: (g, 0)),
            pl.BlockSpec((blk, _N), lambda g: (g, 0)),
            pl.BlockSpec((blk, _N), lambda g: (g, 0)),
        ],
        out_specs=pl.BlockSpec(memory_space=pltpu.SMEM),
        out_shape=jax.ShapeDtypeStruct((1,), jnp.float32),
        scratch_shapes=[
            pltpu.VMEM((_B, _N), jnp.float32),
            pltpu.VMEM((_B, _N), jnp.float32),
            pltpu.VMEM((_B, _N), jnp.float32),
            pltpu.VMEM((_B, _N), jnp.float32),
            pltpu.VMEM((_B, _N), jnp.float32),
        ],
        compiler_params=pltpu.CompilerParams(
            dimension_semantics=("arbitrary",),
        ),
    )(jnp.asarray(lam, jnp.float32).reshape(1), ct, st, it)
    return out[0]


def kernel(lam, conf, conf_flip, loc, loc_flip, conf_shuffle,
           conf_interpolation, loc_shuffle, loc_interpolation):
    return _isd_loss(lam, conf, conf_shuffle, conf_interpolation)


# speedup vs baseline: 1.1030x; 1.0147x over previous
import jax
import jax.numpy as jnp
from jax.experimental import pallas as pl
from jax.experimental.pallas import tpu as pltpu

_B, _N, _C = 32, 8732, 21
_H = _B // 2
_CPB = 3
_STEPS = _C // _CPB


def _isd_body(lam_ref, x_ref, s_ref, i_ref, out_ref,
              acc_ref, x0_ref, t0_ref, lmax_ref, rmax_ref):
    g = pl.program_id(0)
    lam = lam_ref[0]

    ncols = [(i * 1152, 1152) for i in range(7)] + [(8064, _N - 8064)]
    for p in range(_CPB):
        for r in range(2):
            rows = pl.ds((2 * p + r) * _H, _H)
            srows = pl.ds((2 * p + (1 - r)) * _H, _H)
            sl = pl.ds(r * _H, _H)
            for c0, cw in ncols:
                cs = pl.ds(c0, cw)

                x = x_ref[rows, cs]
                t = s_ref[srows, cs]
                ci = i_ref[rows, cs]

                mixed = lam * x + ((1.0 - lam) * t + 1e-7)
                interp = ci + 1e-7
                contrib = (interp - mixed) * (jnp.log(interp)
                                              - jnp.log(mixed))

                if p == 0:
                    @pl.when(g == 0)
                    def _(contrib=contrib, x=x, t=t, sl=sl, cs=cs):
                        acc_ref[sl, cs] = contrib
                        x0_ref[sl, cs] = x
                        t0_ref[sl, cs] = t

                    @pl.when(g > 0)
                    def _(contrib=contrib, x=x, t=t, sl=sl, cs=cs):
                        acc_ref[sl, cs] += contrib
                        lmax_ref[sl, cs] = jnp.maximum(lmax_ref[sl, cs], x)
                        rmax_ref[sl, cs] = jnp.maximum(rmax_ref[sl, cs], t)
                elif p == 1:
                    acc_ref[sl, cs] += contrib

                    @pl.when(g == 0)
                    def _(x=x, t=t, sl=sl, cs=cs):
                        lmax_ref[sl, cs] = x
                        rmax_ref[sl, cs] = t

                    @pl.when(g > 0)
                    def _(x=x, t=t, sl=sl, cs=cs):
                        lmax_ref[sl, cs] = jnp.maximum(lmax_ref[sl, cs], x)
                        rmax_ref[sl, cs] = jnp.maximum(rmax_ref[sl, cs], t)
                else:
                    acc_ref[sl, cs] += contrib
                    lmax_ref[sl, cs] = jnp.maximum(lmax_ref[sl, cs], x)
                    rmax_ref[sl, cs] = jnp.maximum(rmax_ref[sl, cs], t)

    @pl.when(g == _STEPS - 1)
    def _():
        mask = jnp.logical_and(lmax_ref[...] > x0_ref[...],
                               rmax_ref[...] > t0_ref[...])
        maskf = mask.astype(jnp.float32)
        s = jnp.sum(acc_ref[...] * maskf)
        cnt = jnp.sum(maskf)
        loss = 0.5 * s / jnp.maximum(cnt, 1.0)
        out_ref[0] = jnp.where(cnt > 0.0, loss, 0.0)


@jax.jit
def _isd_loss(lam, conf, conf_shuffle, conf_interpolation):
    ct = jnp.transpose(conf, (2, 0, 1)).reshape(_C * _B, _N)
    st = jnp.transpose(conf_shuffle, (2, 0, 1)).reshape(_C * _B, _N)
    it = jnp.transpose(conf_interpolation, (2, 0, 1)).reshape(_C * _B, _N)
    blk = _CPB * _B
    out = pl.pallas_call(
        _isd_body,
        grid=(_STEPS,),
        in_specs=[
            pl.BlockSpec(memory_space=pltpu.SMEM),
            pl.BlockSpec((blk, _N), lambda g: (g, 0)),
            pl.BlockSpec((blk, _N), lambda g: (g, 0)),
            pl.BlockSpec((blk, _N), lambda g: (g, 0)),
        ],
        out_specs=pl.BlockSpec(memory_space=pltpu.SMEM),
        out_shape=jax.ShapeDtypeStruct((1,), jnp.float32),
        scratch_shapes=[
            pltpu.VMEM((_B, _N), jnp.float32),
            pltpu.VMEM((_B, _N), jnp.float32),
            pltpu.VMEM((_B, _N), jnp.float32),
            pltpu.VMEM((_B, _N), jnp.float32),
            pltpu.VMEM((_B, _N), jnp.float32),
        ],
        compiler_params=pltpu.CompilerParams(
            dimension_semantics=("arbitrary",),
        ),
    )(jnp.asarray(lam, jnp.float32).reshape(1), ct, st, it)
    return out[0]


def kernel(lam, conf, conf_flip, loc, loc_flip, conf_shuffle,
           conf_interpolation, loc_shuffle, loc_interpolation):
    return _isd_loss(lam, conf, conf_shuffle, conf_interpolation)
